# baseline (speedup 1.0000x reference)
import functools

import numpy as np
import jax
import jax.numpy as jnp
from jax import lax
from jax.experimental import pallas as pl
from jax.experimental.pallas import tpu as pltpu
from jax.experimental.pallas import tpu_sc as plsc

NUM_LEVELS = 12
TABLE_SIZE = 1 << 17
BOUND = 0.15
BASE_RES = 64
DESIRED_RES = 384 * 0.15
N_PTS = 500000
SEQ_LEN = 8

_scale = 2.0 ** (np.log2(DESIRED_RES / BASE_RES) / (NUM_LEVELS - 1))
RES = [int(np.ceil(BASE_RES * _scale ** l)) for l in range(NUM_LEVELS)]
OFF = np.cumsum([0] + [r * r for r in RES]).astype(np.int64)
E = int(OFF[-1])

CHUNK = 128
WAVE = 16
NCHUNKS = -(-E // CHUNK)
NWAVES = NCHUNKS // WAVE
WAVE_REM = NCHUNKS - NWAVES * WAVE
E_PAD = NCHUNKS * CHUNK

BATCH = 2048
N_PAD = 512000
NTILES_PER_PLANE = 10
NACTIVE = 3 * NTILES_PER_PLANE
CHUNK_PTS = N_PAD // NTILES_PER_PLANE
NBATCH = CHUNK_PTS // BATCH
NGRP = BATCH // 16
NBLK = N_PAD // BATCH
BLKW = 36 * BATCH


def _build_gidx():
    base = np.zeros((E_PAD,), np.int64)
    for l, r in enumerate(RES):
        ys, xs = np.meshgrid(np.arange(r, dtype=np.uint32),
                             np.arange(r, dtype=np.uint32), indexing='ij')
        h = (xs ^ (ys * np.uint32(2654435761))) & np.uint32(TABLE_SIZE - 1)
        base[OFF[l]:OFF[l + 1]] = (l * TABLE_SIZE + h.astype(np.int64)).reshape(-1)
    return np.concatenate([(base + p * NUM_LEVELS * TABLE_SIZE).astype(np.int32)
                           for p in range(3)])


_GIDX3 = _build_gidx()


def _sc_encode_body(tbl_hbm, cAB_hbm, gidx_hbm, out_hbm,
                    gidx_v, ctab_v, cab_v, stage_v, sem, sem_in, sem_out):
    c = lax.axis_index("c")
    s = lax.axis_index("s")
    wid = s * 2 + c

    @pl.when(wid < NACTIVE)
    def _():
        p = wid // NTILES_PER_PLANE
        base = (wid % NTILES_PER_PLANE) * CHUNK_PTS
        pstart = p * N_PAD + base
        inv2b = float(2.0 * BOUND)

        def fire_in(ib, slot):
            off = pstart + ib * BATCH
            pltpu.async_copy(cAB_hbm.at[pl.ds(off, BATCH)],
                             cab_v.at[slot, 0], sem_in)
            pltpu.async_copy(cAB_hbm.at[pl.ds(3 * N_PAD + off, BATCH)],
                             cab_v.at[slot, 1], sem_in)

        def drain_in():
            for _i in range(2):
                pltpu.make_async_copy(cAB_hbm.at[pl.ds(0, BATCH)],
                                      cab_v.at[0, 0], sem_in).wait()

        fire_in(0, 0)

        pltpu.sync_copy(gidx_hbm.at[pl.ds(p * E_PAD, E_PAD)], gidx_v)

        def fire_wave(wv, carry):
            def fire(j, carry2):
                k = wv * WAVE + j
                pltpu.async_copy(
                    tbl_hbm.at[gidx_v.at[pl.ds(k * CHUNK, CHUNK)]],
                    ctab_v.at[pl.ds(k * CHUNK, CHUNK)], sem)
                return carry2
            lax.fori_loop(0, WAVE, fire, 0)
            return carry

        def drain_wave():
            pltpu.make_async_copy(
                tbl_hbm.at[pl.ds(0, WAVE * CHUNK)],
                ctab_v.at[pl.ds(0, WAVE * CHUNK)], sem).wait()

        fire_wave(0, 0)

        def wave_body(wv, carry):
            fire_wave(wv + 1, 0)
            drain_wave()
            return carry
        lax.fori_loop(0, NWAVES - 1, wave_body, 0)
        for k in range(NWAVES * WAVE, NCHUNKS):
            pltpu.async_copy(
                tbl_hbm.at[gidx_v.at[pl.ds(k * CHUNK, CHUNK)]],
                ctab_v.at[pl.ds(k * CHUNK, CHUNK)], sem)
        drain_wave()
        for k in range(NWAVES * WAVE, NCHUNKS):
            pltpu.make_async_copy(
                tbl_hbm.at[pl.ds(0, CHUNK)],
                ctab_v.at[pl.ds(0, CHUNK)], sem).wait()

        def batch_body(ib, carry):
            slot = lax.rem(ib, 2)

            @pl.when(ib + 1 < NBATCH)
            def _():
                fire_in(ib + 1, lax.rem(ib + 1, 2))

            drain_in()

            @pl.when(ib > 0)
            def _():
                pltpu.make_async_copy(tbl_hbm.at[pl.ds(0, NUM_LEVELS * BATCH)],
                                      stage_v, sem_out).wait()

            @plsc.parallel_loop(0, NGRP, 1, unroll=8)
            def grp(i):
                ar = cab_v[slot, 0, pl.ds(i * 16, 16)]
                br = cab_v[slot, 1, pl.ds(i * 16, 16)]
                ua = jnp.clip((ar + BOUND) / inv2b, 0.0, 1.0)
                ub = jnp.clip((br + BOUND) / inv2b, 0.0, 1.0)
                for l in range(NUM_LEVELS):
                    r = RES[l]
                    posa = ua * np.float32(r - 1)
                    posb = ub * np.float32(r - 1)
                    p0a = jnp.clip(posa.astype(jnp.int32), 0, r - 2)
                    p0b = jnp.clip(posb.astype(jnp.int32), 0, r - 2)
                    fa = posa - p0a.astype(jnp.float32)
                    fb = posb - p0b.astype(jnp.float32)
                    b00 = p0b * r + p0a + int(OFF[l])
                    t00 = plsc.load_gather(ctab_v, [b00])
                    t01 = plsc.load_gather(ctab_v, [b00 + r])
                    t10 = plsc.load_gather(ctab_v, [b00 + 1])
                    t11 = plsc.load_gather(ctab_v, [b00 + (r + 1)])
                    wa1 = 1.0 - fa
                    wb1 = 1.0 - fb
                    acc = (wa1 * wb1) * t00 + (wa1 * fb) * t01 \
                        + (fa * wb1) * t10 + (fa * fb) * t11
                    stage_v[pl.ds(l * BATCH + i * 16, 16)] = acc

            obase = ((base // BATCH + ib) * 36 + p * NUM_LEVELS) * BATCH
            pltpu.async_copy(stage_v, out_hbm.at[pl.ds(obase, NUM_LEVELS * BATCH)],
                             sem_out)
            return carry
        lax.fori_loop(0, NBATCH, batch_body, 0)
        pltpu.make_async_copy(tbl_hbm.at[pl.ds(0, NUM_LEVELS * BATCH)],
                              stage_v, sem_out).wait()


_sc_encode = functools.partial(
    pl.kernel,
    out_type=jax.ShapeDtypeStruct((36 * N_PAD,), jnp.float32),
    mesh=plsc.VectorSubcoreMesh(core_axis_name="c", subcore_axis_name="s"),
    scratch_types=[
        pltpu.VMEM((E_PAD,), jnp.int32),
        pltpu.VMEM((E_PAD,), jnp.float32),
        pltpu.VMEM((2, 2, BATCH), jnp.float32),
        pltpu.VMEM((NUM_LEVELS * BATCH,), jnp.float32),
        pltpu.SemaphoreType.DMA,
        pltpu.SemaphoreType.DMA,
        pltpu.SemaphoreType.DMA,
    ],
    compiler_params=pltpu.CompilerParams(needs_layout_passes=False,
                                         use_tc_tiling_on_sc=False),
)(_sc_encode_body)


def _leaky(v):
    return jnp.where(v >= 0, v, 0.02 * v)


def _audio_body(a2, aW1T, ab1, aW2T, ab2, aW3T, ab3,
                w10, w11, w12, b1r, w20, w21, w22, b2r,
                w30, w31, w32, b3r, w40, w41, w42, b4r,
                w50, w51, w52, b5r, attW, attb,
                o_enca):
    dot = functools.partial(jnp.dot, preferred_element_type=jnp.float32)
    hdot = functools.partial(jnp.dot, preferred_element_type=jnp.float32,
                             precision=lax.Precision.HIGHEST)
    e = _leaky(dot(a2[...], aW1T[...]) + ab1[...])
    e = _leaky(dot(e, aW2T[...]) + ab2[...])
    e3 = dot(e, aW3T[...]) + ab3[...]
    y = e3
    for w0, w1, w2, br in ((w10, w11, w12, b1r), (w20, w21, w22, b2r),
                           (w30, w31, w32, b3r), (w40, w41, w42, b4r),
                           (w50, w51, w52, b5r)):
        z = jnp.zeros((1, y.shape[1]), jnp.float32)
        yp = jnp.concatenate([z, y, z], axis=0)
        y = _leaky(dot(yp[0:8], w0[...]) + dot(yp[1:9], w1[...])
                   + dot(yp[2:10], w2[...]) + br[...])
    lg = hdot(attW[...], y) + attb[...]
    ew = jnp.exp(lg - jnp.max(lg))
    wat = ew / jnp.sum(ew)
    o_enca[...] = jnp.sum(wat * e3, axis=0, keepdims=True)


_audio_call = pl.pallas_call(
    _audio_body,
    out_shape=jax.ShapeDtypeStruct((1, 32), jnp.float32),
)


BN = BATCH


def _mlp_body(enc, S1f, S2, S3, Sc1f, Sc2, Sc3, enca, mvc, o_xyz, o_rot):
    dot = functools.partial(jnp.dot, preferred_element_type=jnp.float32)
    eb = enc[...].reshape(36, BN)
    ew = jnp.broadcast_to(enca[...], (32, BN))
    mv = jnp.broadcast_to(mvc[...], (3, BN))
    h_in = jnp.concatenate([eb, ew, mv], axis=0)
    h = jnp.maximum(dot(S1f[...], h_in), 0.0)
    h = jnp.maximum(dot(S2[...], h), 0.0)
    h = dot(S3[...], h)
    g_in = jnp.concatenate([eb, mv], axis=0)
    g = jnp.maximum(dot(Sc1f[...], g_in), 0.0)
    g = jnp.maximum(dot(Sc2[...], g), 0.0)
    g = dot(Sc3[...], g)
    sg = 1.0 / (1.0 + jnp.exp(-g))
    hx = jnp.concatenate([h[0:1] * 0.002, h[1:2] * 0.01, h[2:3] * 0.002], axis=0)
    o_xyz[...] = hx * (sg * 2.0)
    o_rot[...] = h[3:]


def _full(shape):
    return pl.BlockSpec(shape, lambda i: (0, 0))


_mlp_call = pl.pallas_call(
    _mlp_body,
    grid=(NBLK,),
    in_specs=[
        pl.BlockSpec((BLKW,), lambda i: (i,)),
        _full((32, 71)), _full((32, 32)), _full((7, 32)),
        _full((16, 39)), _full((16, 16)), _full((1, 16)),
        _full((32, 1)), _full((3, 1)),
    ],
    out_specs=(pl.BlockSpec((3, BN), lambda i: (0, i)),
               pl.BlockSpec((4, BN), lambda i: (0, i))),
    out_shape=(jax.ShapeDtypeStruct((3, N_PAD), jnp.float32),
               jax.ShapeDtypeStruct((4, N_PAD), jnp.float32)),
)


def kernel(x, a, move, table_xy, table_yz, table_xz, aW1, ab1, aW2, ab2, aW3, ab3,
           c1w, c1b, c2w, c2b, c3w, c3b, c4w, c4b, c5w, c5b, attW, attb,
           S1, S2, S3, Sc1, Sc2, Sc3):
    tbl = jnp.concatenate([table_xy.reshape(-1), table_yz.reshape(-1),
                           table_xz.reshape(-1)])
    z = jnp.zeros((N_PAD - N_PTS,), jnp.float32)
    cAB = jnp.concatenate([x[:, 0], z, x[:, 1], z, x[:, 0], z,
                           x[:, 1], z, x[:, 2], z, x[:, 2], z])
    enc = _sc_encode(tbl, cAB, jnp.asarray(_GIDX3))

    a2 = a.reshape(SEQ_LEN, 512)
    row = lambda v: v.reshape(1, -1)
    conv_args = []
    for w, b in ((c1w, c1b), (c2w, c2b), (c3w, c3b), (c4w, c4b), (c5w, c5b)):
        conv_args += [w[:, :, 0].T, w[:, :, 1].T, w[:, :, 2].T, row(b)]
    enca = _audio_call(a2, aW1.T, row(ab1), aW2.T, row(ab2), aW3.T, row(ab3),
                       *conv_args, attW, attb.reshape(-1, 1))

    o_xyzT, o_rotT = _mlp_call(enc, S1, S2, S3, Sc1, Sc2, Sc3,
                               enca.reshape(32, 1), move.reshape(3, 1))
    return o_xyzT[:, :N_PTS].T, o_rotT[:, :N_PTS].T


# speedup vs baseline: 1.2555x; 1.2555x over previous
import functools

import numpy as np
import jax
import jax.numpy as jnp
from jax import lax
from jax.experimental import pallas as pl
from jax.experimental.pallas import tpu as pltpu
from jax.experimental.pallas import tpu_sc as plsc

NUM_LEVELS = 12
TABLE_SIZE = 1 << 17
BOUND = 0.15
BASE_RES = 64
DESIRED_RES = 384 * 0.15
N_PTS = 500000
SEQ_LEN = 8

_scale = 2.0 ** (np.log2(DESIRED_RES / BASE_RES) / (NUM_LEVELS - 1))
RES = [int(np.ceil(BASE_RES * _scale ** l)) for l in range(NUM_LEVELS)]
OFF = np.cumsum([0] + [r * r for r in RES]).astype(np.int64)
E = int(OFF[-1])

CHUNK = 128
WAVE = 16
NCHUNKS = -(-E // CHUNK)
NWAVES = NCHUNKS // WAVE
WAVE_REM = NCHUNKS - NWAVES * WAVE
E_PAD = NCHUNKS * CHUNK

BATCH = 2048
N_PAD = 512000
NTILES_PER_PLANE = 10
NACTIVE = 3 * NTILES_PER_PLANE
CHUNK_PTS = N_PAD // NTILES_PER_PLANE
NBATCH = CHUNK_PTS // BATCH
NGRP = BATCH // 16
NBLK = N_PAD // BATCH
BLKW = 36 * BATCH


def _build_gidx():
    base = np.zeros((E_PAD,), np.int64)
    for l, r in enumerate(RES):
        ys, xs = np.meshgrid(np.arange(r, dtype=np.uint32),
                             np.arange(r, dtype=np.uint32), indexing='ij')
        h = (xs ^ (ys * np.uint32(2654435761))) & np.uint32(TABLE_SIZE - 1)
        base[OFF[l]:OFF[l + 1]] = (l * TABLE_SIZE + h.astype(np.int64)).reshape(-1)
    return np.concatenate([(base + p * NUM_LEVELS * TABLE_SIZE).astype(np.int32)
                           for p in range(3)])


_GIDX3 = _build_gidx()


def _sc_encode_body(tbl_hbm, cAB_hbm, gidx_hbm, out_hbm,
                    gidx_v, ctab_v, cab_v, stage_v, sem, sem_in, sem_out):
    c = lax.axis_index("c")
    s = lax.axis_index("s")
    wid = s * 2 + c

    @pl.when(wid < NACTIVE)
    def _():
        p = wid // NTILES_PER_PLANE
        base = (wid % NTILES_PER_PLANE) * CHUNK_PTS
        pstart = p * N_PAD + base
        inv2b = float(2.0 * BOUND)

        def fire_in(ib, slot):
            off = pstart + ib * BATCH
            pltpu.async_copy(cAB_hbm.at[pl.ds(off, BATCH)],
                             cab_v.at[slot, 0], sem_in)
            pltpu.async_copy(cAB_hbm.at[pl.ds(3 * N_PAD + off, BATCH)],
                             cab_v.at[slot, 1], sem_in)

        def drain_in():
            for _i in range(2):
                pltpu.make_async_copy(cAB_hbm.at[pl.ds(0, BATCH)],
                                      cab_v.at[0, 0], sem_in).wait()

        fire_in(0, 0)

        pltpu.sync_copy(gidx_hbm.at[pl.ds(p * E_PAD, E_PAD)], gidx_v)

        def fire_wave(wv, carry):
            def fire(j, carry2):
                k = wv * WAVE + j
                pltpu.async_copy(
                    tbl_hbm.at[gidx_v.at[pl.ds(k * CHUNK, CHUNK)]],
                    ctab_v.at[pl.ds(k * CHUNK, CHUNK)], sem)
                return carry2
            lax.fori_loop(0, WAVE, fire, 0)
            return carry

        def drain_wave():
            pltpu.make_async_copy(
                tbl_hbm.at[pl.ds(0, WAVE * CHUNK)],
                ctab_v.at[pl.ds(0, WAVE * CHUNK)], sem).wait()

        fire_wave(0, 0)

        def wave_body(wv, carry):
            fire_wave(wv + 1, 0)
            drain_wave()
            return carry
        lax.fori_loop(0, NWAVES - 1, wave_body, 0)
        for k in range(NWAVES * WAVE, NCHUNKS):
            pltpu.async_copy(
                tbl_hbm.at[gidx_v.at[pl.ds(k * CHUNK, CHUNK)]],
                ctab_v.at[pl.ds(k * CHUNK, CHUNK)], sem)
        drain_wave()
        for k in range(NWAVES * WAVE, NCHUNKS):
            pltpu.make_async_copy(
                tbl_hbm.at[pl.ds(0, CHUNK)],
                ctab_v.at[pl.ds(0, CHUNK)], sem).wait()

        def batch_body(ib, carry):
            slot = lax.rem(ib, 2)

            @pl.when(ib + 1 < NBATCH)
            def _():
                fire_in(ib + 1, lax.rem(ib + 1, 2))

            drain_in()

            @pl.when(ib > 0)
            def _():
                pltpu.make_async_copy(tbl_hbm.at[pl.ds(0, NUM_LEVELS * BATCH)],
                                      stage_v, sem_out).wait()

            @plsc.parallel_loop(0, NGRP, 1, unroll=8)
            def grp(i):
                ar = cab_v[slot, 0, pl.ds(i * 16, 16)]
                br = cab_v[slot, 1, pl.ds(i * 16, 16)]
                ua = jnp.clip((ar + BOUND) / inv2b, 0.0, 1.0)
                ub = jnp.clip((br + BOUND) / inv2b, 0.0, 1.0)
                for l in range(NUM_LEVELS):
                    r = RES[l]
                    posa = ua * np.float32(r - 1)
                    posb = ub * np.float32(r - 1)
                    p0a = jnp.clip(posa.astype(jnp.int32), 0, r - 2)
                    p0b = jnp.clip(posb.astype(jnp.int32), 0, r - 2)
                    fa = posa - p0a.astype(jnp.float32)
                    fb = posb - p0b.astype(jnp.float32)
                    b00 = p0b * r + p0a + int(OFF[l])
                    t00 = plsc.load_gather(ctab_v, [b00])
                    t01 = plsc.load_gather(ctab_v, [b00 + r])
                    t10 = plsc.load_gather(ctab_v, [b00 + 1])
                    t11 = plsc.load_gather(ctab_v, [b00 + (r + 1)])
                    wa1 = 1.0 - fa
                    wb1 = 1.0 - fb
                    acc = (wa1 * wb1) * t00 + (wa1 * fb) * t01 \
                        + (fa * wb1) * t10 + (fa * fb) * t11
                    stage_v[pl.ds(l * BATCH + i * 16, 16)] = acc

            obase = ((base // BATCH + ib) * 36 + p * NUM_LEVELS) * BATCH
            pltpu.async_copy(stage_v, out_hbm.at[pl.ds(obase, NUM_LEVELS * BATCH)],
                             sem_out)
            return carry
        lax.fori_loop(0, NBATCH, batch_body, 0)
        pltpu.make_async_copy(tbl_hbm.at[pl.ds(0, NUM_LEVELS * BATCH)],
                              stage_v, sem_out).wait()


_sc_encode = functools.partial(
    pl.kernel,
    out_type=jax.ShapeDtypeStruct((36 * N_PAD,), jnp.float32),
    mesh=plsc.VectorSubcoreMesh(core_axis_name="c", subcore_axis_name="s"),
    scratch_types=[
        pltpu.VMEM((E_PAD,), jnp.int32),
        pltpu.VMEM((E_PAD,), jnp.float32),
        pltpu.VMEM((2, 2, BATCH), jnp.float32),
        pltpu.VMEM((NUM_LEVELS * BATCH,), jnp.float32),
        pltpu.SemaphoreType.DMA,
        pltpu.SemaphoreType.DMA,
        pltpu.SemaphoreType.DMA,
    ],
    compiler_params=pltpu.CompilerParams(needs_layout_passes=False,
                                         use_tc_tiling_on_sc=False),
)(_sc_encode_body)


def _leaky(v):
    return jnp.where(v >= 0, v, 0.02 * v)


def _audio_body(a2, aW1T, ab1, aW2T, ab2, aW3T, ab3,
                w10, w11, w12, b1r, w20, w21, w22, b2r,
                w30, w31, w32, b3r, w40, w41, w42, b4r,
                w50, w51, w52, b5r, attW, attb,
                o_enca):
    dot = functools.partial(jnp.dot, preferred_element_type=jnp.float32)
    hdot = functools.partial(jnp.dot, preferred_element_type=jnp.float32,
                             precision=lax.Precision.HIGHEST)
    e = _leaky(dot(a2[...], aW1T[...]) + ab1[...])
    e = _leaky(dot(e, aW2T[...]) + ab2[...])
    e3 = dot(e, aW3T[...]) + ab3[...]
    y = e3
    for w0, w1, w2, br in ((w10, w11, w12, b1r), (w20, w21, w22, b2r),
                           (w30, w31, w32, b3r), (w40, w41, w42, b4r),
                           (w50, w51, w52, b5r)):
        z = jnp.zeros((1, y.shape[1]), jnp.float32)
        yp = jnp.concatenate([z, y, z], axis=0)
        y = _leaky(dot(yp[0:8], w0[...]) + dot(yp[1:9], w1[...])
                   + dot(yp[2:10], w2[...]) + br[...])
    lg = hdot(attW[...], y) + attb[...]
    ew = jnp.exp(lg - jnp.max(lg))
    wat = ew / jnp.sum(ew)
    o_enca[...] = jnp.sum(wat * e3, axis=0, keepdims=True)


_audio_call = pl.pallas_call(
    _audio_body,
    out_shape=jax.ShapeDtypeStruct((1, 32), jnp.float32),
)


NSUB = 5
BN = NSUB * BATCH
NBLK_MLP = N_PAD // BN


def _mlp_body(enc, S1f, S2, S3, Sc1f, Sc2, Sc3, enca, mvc, o_xyz, o_rot):
    dot = functools.partial(jnp.dot, preferred_element_type=jnp.float32)
    e5 = enc[...].reshape(NSUB, 36, BATCH)
    eb = jnp.concatenate([e5[k] for k in range(NSUB)], axis=1)
    ew = jnp.broadcast_to(enca[...], (32, BN))
    mv = jnp.broadcast_to(mvc[...], (3, BN))
    h_in = jnp.concatenate([eb, ew, mv], axis=0)
    h = jnp.maximum(dot(S1f[...], h_in), 0.0)
    h = jnp.maximum(dot(S2[...], h), 0.0)
    h = dot(S3[...], h)
    g_in = jnp.concatenate([eb, mv], axis=0)
    g = jnp.maximum(dot(Sc1f[...], g_in), 0.0)
    g = jnp.maximum(dot(Sc2[...], g), 0.0)
    g = dot(Sc3[...], g)
    sg = 1.0 / (1.0 + jnp.exp(-g))
    hx = jnp.concatenate([h[0:1] * 0.002, h[1:2] * 0.01, h[2:3] * 0.002], axis=0)
    o_xyz[...] = hx * (sg * 2.0)
    o_rot[...] = h[3:]


def _full(shape):
    return pl.BlockSpec(shape, lambda i: (0, 0))


_mlp_call = pl.pallas_call(
    _mlp_body,
    grid=(NBLK_MLP,),
    in_specs=[
        pl.BlockSpec((NSUB * BLKW,), lambda i: (i,)),
        _full((32, 71)), _full((32, 32)), _full((7, 32)),
        _full((16, 39)), _full((16, 16)), _full((1, 16)),
        _full((32, 1)), _full((3, 1)),
    ],
    out_specs=(pl.BlockSpec((3, BN), lambda i: (0, i)),
               pl.BlockSpec((4, BN), lambda i: (0, i))),
    out_shape=(jax.ShapeDtypeStruct((3, N_PAD), jnp.float32),
               jax.ShapeDtypeStruct((4, N_PAD), jnp.float32)),
)


def kernel(x, a, move, table_xy, table_yz, table_xz, aW1, ab1, aW2, ab2, aW3, ab3,
           c1w, c1b, c2w, c2b, c3w, c3b, c4w, c4b, c5w, c5b, attW, attb,
           S1, S2, S3, Sc1, Sc2, Sc3):
    tbl = jnp.concatenate([table_xy.reshape(-1), table_yz.reshape(-1),
                           table_xz.reshape(-1)])
    z = jnp.zeros((N_PAD - N_PTS,), jnp.float32)
    cAB = jnp.concatenate([x[:, 0], z, x[:, 1], z, x[:, 0], z,
                           x[:, 1], z, x[:, 2], z, x[:, 2], z])
    enc = _sc_encode(tbl, cAB, jnp.asarray(_GIDX3))

    a2 = a.reshape(SEQ_LEN, 512)
    row = lambda v: v.reshape(1, -1)
    conv_args = []
    for w, b in ((c1w, c1b), (c2w, c2b), (c3w, c3b), (c4w, c4b), (c5w, c5b)):
        conv_args += [w[:, :, 0].T, w[:, :, 1].T, w[:, :, 2].T, row(b)]
    enca = _audio_call(a2, aW1.T, row(ab1), aW2.T, row(ab2), aW3.T, row(ab3),
                       *conv_args, attW, attb.reshape(-1, 1))

    o_xyzT, o_rotT = _mlp_call(enc, S1, S2, S3, Sc1, Sc2, Sc3,
                               enca.reshape(32, 1), move.reshape(3, 1))
    return o_xyzT[:, :N_PTS].T, o_rotT[:, :N_PTS].T
